# baseline (speedup 1.0000x reference)
import functools

import jax
import jax.numpy as jnp
from jax import lax
from jax.experimental import pallas as pl
from jax.experimental.pallas import tpu as pltpu
from jax.experimental.pallas import tpu_sc as plsc

_R, _C = 4096, 4096
_S = 1536
_NW = 32
_ROWS_W = _S // _NW
_CHROWS = 8
_NCH = _ROWS_W // _CHROWS
_NBUF = 3
_UNROLL = 8
_TCBLK = 512

_mesh = plsc.VectorSubcoreMesh(core_axis_name="c", subcore_axis_name="s")


@functools.partial(
    pl.kernel,
    mesh=_mesh,
    out_type=jax.ShapeDtypeStruct((_S, _C), jnp.float32),
    scratch_types=[pltpu.VMEM((_CHROWS, _C), jnp.float32)] * _NBUF
    + [
        pltpu.VMEM((16,), jnp.float32),
        pltpu.SemaphoreType.DMA,
        pltpu.SemaphoreType.DMA,
    ],
)
def _sc_scale(w_hbm, m_hbm, out_hbm, *rest):
    bufs = rest[:_NBUF]
    mvec, sem_in, sem_out = rest[_NBUF:]
    wid = lax.axis_index("s") * 2 + lax.axis_index("c")
    base = wid * _ROWS_W

    pltpu.sync_copy(m_hbm, mvec)
    mv = mvec[...]

    def compute(buf):
        for r in range(_CHROWS):
            @plsc.parallel_loop(0, _C, 16, unroll=_UNROLL)
            def _(c):
                sl = pl.ds(c, 16)
                buf[r, sl] = buf[r, sl] * mv

    def start_in(i):
        return pltpu.async_copy(
            w_hbm.at[pl.ds(base + i * _CHROWS, _CHROWS)], bufs[i % _NBUF], sem_in)

    def start_out(i):
        return pltpu.async_copy(
            bufs[i % _NBUF], out_hbm.at[pl.ds(base + i * _CHROWS, _CHROWS)], sem_out)

    out_cp = [None] * _NCH
    in_cp = [None] * _NCH
    out_waited = [False] * _NCH
    for j in range(_NBUF - 1):
        in_cp[j] = start_in(j)
    for i in range(_NCH):
        nxt = i + _NBUF - 1
        if nxt < _NCH:
            if nxt >= _NBUF:
                out_cp[nxt - _NBUF].wait()
                out_waited[nxt - _NBUF] = True
            in_cp[nxt] = start_in(nxt)
        in_cp[i].wait()
        compute(bufs[i % _NBUF])
        out_cp[i] = start_out(i)
    for i in range(_NCH):
        if not out_waited[i]:
            out_cp[i].wait()


def _tc_body(m_ref, w_ref, o_ref):
    o_ref[...] = w_ref[...] * m_ref[0]


def _tc_scale(weight, m1):
    nblk = (_R - _S) // _TCBLK
    return pl.pallas_call(
        _tc_body,
        grid=(nblk,),
        in_specs=[
            pl.BlockSpec(memory_space=pltpu.SMEM),
            pl.BlockSpec((_TCBLK, _C), lambda i: (i + _S // _TCBLK, 0)),
        ],
        out_specs=pl.BlockSpec((_TCBLK, _C), lambda i: (i + _S // _TCBLK, 0)),
        out_shape=jax.ShapeDtypeStruct((_R, _C), jnp.float32),
    )(m1, weight)


def kernel(weight, mask):
    m1 = jnp.reshape(mask.astype(jnp.float32), (1,))
    m16 = jnp.broadcast_to(m1, (16,))
    top = _sc_scale(weight, m16)
    full = _tc_scale(weight, m1)
    return lax.dynamic_update_slice(full, top, (0, 0))


# speedup vs baseline: 1.1645x; 1.1645x over previous
import functools

import jax
import jax.numpy as jnp
from jax import lax
from jax.experimental import pallas as pl
from jax.experimental.pallas import tpu as pltpu
from jax.experimental.pallas import tpu_sc as plsc

_R, _C = 4096, 4096
_S = 512
_NW = 32
_ROWS_W = _S // _NW
_CHROWS = 4
_NCH = _ROWS_W // _CHROWS
_NBUF = 3
_UNROLL = 8
_TCBLK = 512

_mesh = plsc.VectorSubcoreMesh(core_axis_name="c", subcore_axis_name="s")


@functools.partial(
    pl.kernel,
    mesh=_mesh,
    out_type=jax.ShapeDtypeStruct((_S, _C), jnp.float32),
    scratch_types=[pltpu.VMEM((_CHROWS, _C), jnp.float32)] * _NBUF
    + [
        pltpu.VMEM((16,), jnp.float32),
        pltpu.SemaphoreType.DMA,
        pltpu.SemaphoreType.DMA,
    ],
)
def _sc_scale(w_hbm, m_hbm, out_hbm, *rest):
    bufs = rest[:_NBUF]
    mvec, sem_in, sem_out = rest[_NBUF:]
    wid = lax.axis_index("s") * 2 + lax.axis_index("c")
    base = wid * _ROWS_W

    pltpu.sync_copy(m_hbm, mvec)
    mv = mvec[...]

    def compute(buf):
        for r in range(_CHROWS):
            @plsc.parallel_loop(0, _C, 16, unroll=_UNROLL)
            def _(c):
                sl = pl.ds(c, 16)
                buf[r, sl] = buf[r, sl] * mv

    def start_in(i):
        return pltpu.async_copy(
            w_hbm.at[pl.ds(base + i * _CHROWS, _CHROWS)], bufs[i % _NBUF], sem_in)

    def start_out(i):
        return pltpu.async_copy(
            bufs[i % _NBUF], out_hbm.at[pl.ds(base + i * _CHROWS, _CHROWS)], sem_out)

    out_cp = [None] * _NCH
    in_cp = [None] * _NCH
    out_waited = [False] * _NCH
    for j in range(_NBUF - 1):
        in_cp[j] = start_in(j)
    for i in range(_NCH):
        nxt = i + _NBUF - 1
        if nxt < _NCH:
            if nxt >= _NBUF:
                out_cp[nxt - _NBUF].wait()
                out_waited[nxt - _NBUF] = True
            in_cp[nxt] = start_in(nxt)
        in_cp[i].wait()
        compute(bufs[i % _NBUF])
        out_cp[i] = start_out(i)
    for i in range(_NCH):
        if not out_waited[i]:
            out_cp[i].wait()


def _tc_body(m_ref, w_ref, o_ref):
    o_ref[...] = w_ref[...] * m_ref[0]


def _tc_scale(weight, m1):
    nblk = (_R - _S) // _TCBLK
    return pl.pallas_call(
        _tc_body,
        grid=(nblk,),
        in_specs=[
            pl.BlockSpec(memory_space=pltpu.SMEM),
            pl.BlockSpec((_TCBLK, _C), lambda i: (i + _S // _TCBLK, 0)),
        ],
        out_specs=pl.BlockSpec((_TCBLK, _C), lambda i: (i + _S // _TCBLK, 0)),
        out_shape=jax.ShapeDtypeStruct((_R, _C), jnp.float32),
    )(m1, weight)


def kernel(weight, mask):
    m1 = jnp.reshape(mask.astype(jnp.float32), (1,))
    m16 = jnp.broadcast_to(m1, (16,))
    full = _tc_scale(weight, m1)
    top = _sc_scale(weight, m16)
    return lax.dynamic_update_slice(full, top, (0, 0))
